# baseline (speedup 1.0000x reference)
import jax
import jax.numpy as jnp
import numpy as np
from jax import lax
from jax.experimental import pallas as pl
from jax.experimental.pallas import tpu as pltpu
from jax.experimental.pallas import tpu_sc as plsc

N = 10000
E = 320000
D_IN = 128
D_OUT = 16

NC, NS, LANES = 2, 16, 16
CHUNK = 512
RPC = 4
NCH0, NCH1 = 21, 19
EPAD = NS * (NCH0 + NCH1) * CHUNK
BR = 2000
GRID = N // BR
EB = E // GRID
EPB = EPAD // GRID
PADB = EPB - EB


def _dense_body(x_ref, w_ref, as_ref, ad_ref, ei_ref,
                h_ref, asad_ref, bnd_ref, src_ref, dst_ref):
    i = pl.program_id(0)
    h = jnp.dot(x_ref[...], w_ref[...], preferred_element_type=jnp.float32)
    h_ref[...] = h
    att2 = jnp.stack([as_ref[...], ad_ref[...]], axis=1)
    a2 = jnp.dot(h, att2, preferred_element_type=jnp.float32)
    asad_ref[...] = a2
    mas = jnp.max(a2[:, 0])
    mad = jnp.max(a2[:, 1])
    cur = jnp.stack([jnp.full((128,), mas), jnp.full((128,), mad)])

    @pl.when(i == 0)
    def _init():
        bnd_ref[...] = cur

    @pl.when(i > 0)
    def _acc():
        bnd_ref[...] = jnp.maximum(bnd_ref[...], cur)

    src_ref[pl.ds(0, EB)] = ei_ref[0, :]
    src_ref[pl.ds(EB, PADB)] = jnp.zeros((PADB,), jnp.int32)
    dst_ref[pl.ds(0, EB)] = ei_ref[1, :]
    dst_ref[pl.ds(EB, PADB)] = jnp.zeros((PADB,), jnp.int32)


_dense = pl.pallas_call(
    _dense_body,
    grid=(GRID,),
    in_specs=[
        pl.BlockSpec((BR, D_IN), lambda i: (i, 0)),
        pl.BlockSpec((D_IN, D_OUT), lambda i: (0, 0)),
        pl.BlockSpec((D_OUT,), lambda i: (0,)),
        pl.BlockSpec((D_OUT,), lambda i: (0,)),
        pl.BlockSpec((2, EB), lambda i: (0, i)),
    ],
    out_specs=[
        pl.BlockSpec((BR, D_OUT), lambda i: (i, 0)),
        pl.BlockSpec((BR, 2), lambda i: (i, 0)),
        pl.BlockSpec((2, 128), lambda i: (0, 0)),
        pl.BlockSpec((EPB,), lambda i: (i,)),
        pl.BlockSpec((EPB,), lambda i: (i,)),
    ],
    out_shape=[
        jax.ShapeDtypeStruct((N, D_OUT), jnp.float32),
        jax.ShapeDtypeStruct((N, 2), jnp.float32),
        jax.ShapeDtypeStruct((2, 128), jnp.float32),
        jax.ShapeDtypeStruct((EPAD,), jnp.int32),
        jax.ShapeDtypeStruct((EPAD,), jnp.int32),
    ],
)


_mesh = plsc.VectorSubcoreMesh(
    core_axis_name="c", subcore_axis_name="s", num_cores=NC, num_subcores=NS
)


def _sc_kernel_def(fn):
    return pl.kernel(
        fn,
        out_type=(
            jax.ShapeDtypeStruct((NC * N, D_OUT), jnp.float32),
            jax.ShapeDtypeStruct((NC * N,), jnp.float32),
        ),
        mesh=_mesh,
        compiler_params=pltpu.CompilerParams(
            needs_layout_passes=False, use_tc_tiling_on_sc=False
        ),
        scratch_types=[
            pltpu.VMEM((N, 2), jnp.float32),
            pltpu.VMEM((CHUNK,), jnp.int32),
            pltpu.VMEM((CHUNK,), jnp.int32),
            pltpu.VMEM((CHUNK,), jnp.int32),
            pltpu.VMEM((CHUNK,), jnp.int32),
            pltpu.VMEM((CHUNK,), jnp.float32),
            pltpu.VMEM((CHUNK,), jnp.float32),
            pltpu.VMEM((CHUNK, D_OUT), jnp.float32),
            pltpu.VMEM((CHUNK, D_OUT), jnp.float32),
            pltpu.VMEM((1024,), jnp.float32),
            pltpu.VMEM((2, 128), jnp.float32),
            pltpu.VMEM_SHARED((N, D_OUT), jnp.float32),
            pltpu.VMEM_SHARED((N,), jnp.float32),
            pltpu.VMEM_SHARED((N, D_OUT), jnp.float32),
            pltpu.SemaphoreType.DMA,
            pltpu.SemaphoreType.DMA,
            pltpu.SemaphoreType.DMA,
        ],
    )


@_sc_kernel_def
def _edge_sc(h_hbm, aa_hbm, bnd_hbm, src_hbm, dst_hbm, s_out, d_out,
             aa_v, src_a, src_b, dst_a, dst_b, ex_a, ex_b, hr_a, hr_b,
             zden, bnd_v, s_sh, d_sh, h_sh, gsem, ssem, isem):
    cid = lax.axis_index("c")
    sid = lax.axis_index("s")
    srcb, dstb, exb, hb = [src_a, src_b], [dst_a, dst_b], [ex_a, ex_b], [hr_a, hr_b]

    pltpu.sync_copy(aa_hbm, aa_v)
    pltpu.sync_copy(bnd_hbm, bnd_v)

    braw = bnd_v[0, pl.ds(0, LANES)][0] + bnd_v[1, pl.ds(0, LANES)][0]
    bound = jnp.where(braw > 0.0, braw, 0.2 * braw)

    def _zrow(i, _):
        hr_a[i, :] = jnp.zeros((LANES,), jnp.float32)
        return 0
    lax.fori_loop(0, CHUNK, _zrow, 0)

    def _zden(i, _):
        zden[pl.ds(i * LANES, LANES)] = jnp.zeros((LANES,), jnp.float32)
        return 0
    lax.fori_loop(0, 1024 // LANES, _zden, 0)

    @pl.when(sid < 10)
    def _zero_shared():
        base = sid * 1000
        pltpu.sync_copy(hr_a.at[pl.ds(0, 500)], s_sh.at[pl.ds(base, 500)])
        pltpu.sync_copy(hr_a.at[pl.ds(0, 500)],
                        s_sh.at[pl.ds(base + 500, 500)])
        pltpu.sync_copy(zden.at[pl.ds(0, 1000)], d_sh.at[pl.ds(base, 1000)])

    @pl.when(sid >= 6)
    def _stage_h():
        base = (sid - 6) * 1000
        pltpu.sync_copy(h_hbm.at[pl.ds(base, 1000)],
                        h_sh.at[pl.ds(base, 1000)])

    plsc.subcore_barrier()

    col0 = jnp.zeros((LANES,), jnp.int32)
    col1 = jnp.ones((LANES,), jnp.int32)
    nch = jnp.where(cid == 0, NCH0, NCH1)
    cbase = jnp.where(cid == 0, sid * NCH0, NS * NCH0 + sid * NCH1)
    ebases = [(cbase + k) * CHUNK for k in range(NCH0)]

    def idx_descs(k):
        eb, b = ebases[k], k % 2
        return [
            pltpu.make_async_copy(src_hbm.at[pl.ds(eb, CHUNK)], srcb[b], isem),
            pltpu.make_async_copy(dst_hbm.at[pl.ds(eb, CHUNK)], dstb[b], isem),
        ]

    def gat_descs(k):
        b = k % 2
        src_tab = h_hbm if k % 3 == 0 else h_sh
        return [pltpu.make_async_copy(src_tab.at[srcb[b]], hb[b], gsem)]

    def sc_descs(k):
        b = k % 2
        return [
            pltpu.make_async_copy(hb[b], s_sh.at[dstb[b]], ssem),
            pltpu.make_async_copy(exb[b], d_sh.at[dstb[b]], ssem),
        ]

    def compute_ex(k):
        eb, b = ebases[k], k % 2

        def _exbody(i, _):
            c = i * LANES
            s16 = srcb[b][pl.ds(c, LANES)]
            d16 = dstb[b][pl.ds(c, LANES)]
            e = (plsc.load_gather(aa_v, [s16, col0])
                 + plsc.load_gather(aa_v, [d16, col1]))
            e = jnp.where(e > 0.0, e, 0.2 * e)
            ex = jnp.exp(e - bound)
            eid = eb + c + lax.iota(jnp.int32, 16)
            ex = jnp.where(eid % EPB < EB, ex, 0.0)
            exb[b][pl.ds(c, LANES)] = ex
            return 0
        lax.fori_loop(0, CHUNK // LANES, _exbody, 0)

    def scale(k):
        b = k % 2

        def _sbody(g, _):
            base = g * LANES
            ex16 = exb[b][pl.ds(base, LANES)]
            for l in range(LANES):
                hb[b][base + l, :] = hb[b][base + l, :] * ex16[l]
            return 0
        lax.fori_loop(0, CHUNK // LANES, _sbody, 0)

    for d in idx_descs(0):
        d.start()
    for d in idx_descs(0):
        d.wait()
    for d in gat_descs(0):
        d.start()

    for k in range(NCH0):
        @pl.when(k < nch)
        def _ex(k=k):
            compute_ex(k)

        if k >= 1:
            @pl.when(k - 1 < nch)
            def _drain_sc(k=k):
                for d in sc_descs(k - 1):
                    d.wait()

        if k + 1 < NCH0:
            @pl.when(k + 1 < nch)
            def _fire_idx(k=k):
                for d in idx_descs(k + 1):
                    d.start()

        @pl.when(k < nch)
        def _gath_scale(k=k):
            for d in gat_descs(k):
                d.wait()
            scale(k)

        if k + 1 < NCH0:
            @pl.when(k + 1 < nch)
            def _fire_gat(k=k):
                for d in idx_descs(k + 1):
                    d.wait()
                for d in gat_descs(k + 1):
                    d.start()

        @pl.when(k < nch)
        def _fire_sc(k=k):
            for d in sc_descs(k):
                d.start(add=True)

    @pl.when(NCH0 - 1 < nch)
    def _drain_last():
        for d in sc_descs(NCH0 - 1):
            d.wait()

    plsc.subcore_barrier()

    @pl.when(sid < 10)
    def _copy_out():
        base = sid * 1000
        pltpu.sync_copy(s_sh.at[pl.ds(base, 1000)],
                        s_out.at[pl.ds(cid * N + base, 1000)])
        pltpu.sync_copy(d_sh.at[pl.ds(base, 1000)],
                        d_out.at[pl.ds(cid * N + base, 1000)])


_EXPAND = np.kron(np.eye(8, dtype=np.float32),
                  np.ones((1, D_OUT), np.float32))


def _fin_body(s_ref, d_ref, b_ref, e_ref, o_ref):
    s = s_ref[0] + s_ref[1]
    den8 = d_ref[0] + d_ref[1]
    den = jnp.dot(den8, e_ref[...], preferred_element_type=jnp.float32)
    o_ref[...] = s / (den + 1e-16) + b_ref[...]


_fin = pl.pallas_call(
    _fin_body,
    out_shape=jax.ShapeDtypeStruct((N // 8, 128), jnp.float32),
)


def kernel(x, edge_index, W, att_src, att_dst, bias):
    ei = edge_index.astype(jnp.int32)
    h, asad, bnd, src_p, dst_p = _dense(x, W, att_src, att_dst, ei)
    s_flat, d_flat = _edge_sc(h, asad, bnd, src_p, dst_p)
    out128 = _fin(s_flat.reshape(NC, N // 8, 128),
                  d_flat.reshape(NC, N // 8, 8),
                  jnp.tile(bias, 8).reshape(1, 128),
                  jnp.asarray(_EXPAND))
    return out128.reshape(N, D_OUT)


# speedup vs baseline: 1.0247x; 1.0247x over previous
import jax
import jax.numpy as jnp
import numpy as np
from jax import lax
from jax.experimental import pallas as pl
from jax.experimental.pallas import tpu as pltpu
from jax.experimental.pallas import tpu_sc as plsc

N = 10000
E = 320000
D_IN = 128
D_OUT = 16

NC, NS, LANES = 2, 16, 16
CHUNK = 512
RPC = 4
NCH0, NCH1 = 21, 19
EPAD = NS * (NCH0 + NCH1) * CHUNK
BR = 2000
GRID = N // BR
EB = E // GRID
EPB = EPAD // GRID
PADB = EPB - EB


def _dense_body(x_ref, w_ref, as_ref, ad_ref, ei_ref,
                h_ref, asad_ref, bnd_ref, src_ref, dst_ref):
    i = pl.program_id(0)
    h = jnp.dot(x_ref[...], w_ref[...], preferred_element_type=jnp.float32)
    h_ref[...] = h
    att2 = jnp.stack([as_ref[...], ad_ref[...]], axis=1)
    a2 = jnp.dot(h, att2, preferred_element_type=jnp.float32)
    asad_ref[...] = a2
    mas = jnp.max(a2[:, 0])
    mad = jnp.max(a2[:, 1])
    cur = jnp.stack([jnp.full((128,), mas), jnp.full((128,), mad)])

    @pl.when(i == 0)
    def _init():
        bnd_ref[...] = cur

    @pl.when(i > 0)
    def _acc():
        bnd_ref[...] = jnp.maximum(bnd_ref[...], cur)

    src_ref[pl.ds(0, EB)] = ei_ref[0, :]
    src_ref[pl.ds(EB, PADB)] = jnp.zeros((PADB,), jnp.int32)
    dst_ref[pl.ds(0, EB)] = ei_ref[1, :]
    dst_ref[pl.ds(EB, PADB)] = jnp.zeros((PADB,), jnp.int32)


_dense = pl.pallas_call(
    _dense_body,
    grid=(GRID,),
    in_specs=[
        pl.BlockSpec((BR, D_IN), lambda i: (i, 0)),
        pl.BlockSpec((D_IN, D_OUT), lambda i: (0, 0)),
        pl.BlockSpec((D_OUT,), lambda i: (0,)),
        pl.BlockSpec((D_OUT,), lambda i: (0,)),
        pl.BlockSpec((2, EB), lambda i: (0, i)),
    ],
    out_specs=[
        pl.BlockSpec((BR, D_OUT), lambda i: (i, 0)),
        pl.BlockSpec((BR, 2), lambda i: (i, 0)),
        pl.BlockSpec((2, 128), lambda i: (0, 0)),
        pl.BlockSpec((EPB,), lambda i: (i,)),
        pl.BlockSpec((EPB,), lambda i: (i,)),
    ],
    out_shape=[
        jax.ShapeDtypeStruct((N, D_OUT), jnp.float32),
        jax.ShapeDtypeStruct((N, 2), jnp.float32),
        jax.ShapeDtypeStruct((2, 128), jnp.float32),
        jax.ShapeDtypeStruct((EPAD,), jnp.int32),
        jax.ShapeDtypeStruct((EPAD,), jnp.int32),
    ],
)


_mesh = plsc.VectorSubcoreMesh(
    core_axis_name="c", subcore_axis_name="s", num_cores=NC, num_subcores=NS
)


def _sc_kernel_def(fn):
    return pl.kernel(
        fn,
        out_type=(
            jax.ShapeDtypeStruct((NC * N, D_OUT), jnp.float32),
            jax.ShapeDtypeStruct((NC * N,), jnp.float32),
        ),
        mesh=_mesh,
        compiler_params=pltpu.CompilerParams(
            needs_layout_passes=False, use_tc_tiling_on_sc=False
        ),
        scratch_types=[
            pltpu.VMEM((N, 2), jnp.float32),
            pltpu.VMEM((CHUNK,), jnp.int32),
            pltpu.VMEM((CHUNK,), jnp.int32),
            pltpu.VMEM((CHUNK,), jnp.int32),
            pltpu.VMEM((CHUNK,), jnp.int32),
            pltpu.VMEM((CHUNK,), jnp.float32),
            pltpu.VMEM((CHUNK,), jnp.float32),
            pltpu.VMEM((CHUNK, D_OUT), jnp.float32),
            pltpu.VMEM((CHUNK, D_OUT), jnp.float32),
            pltpu.VMEM((1024,), jnp.float32),
            pltpu.VMEM((2, 128), jnp.float32),
            pltpu.VMEM_SHARED((N, D_OUT), jnp.float32),
            pltpu.VMEM_SHARED((N,), jnp.float32),
            pltpu.VMEM_SHARED((N, D_OUT), jnp.float32),
            pltpu.SemaphoreType.DMA,
            pltpu.SemaphoreType.DMA,
            pltpu.SemaphoreType.DMA,
        ],
    )


@_sc_kernel_def
def _edge_sc(h_hbm, aa_hbm, bnd_hbm, src_hbm, dst_hbm, s_out, d_out,
             aa_v, src_a, src_b, dst_a, dst_b, ex_a, ex_b, hr_a, hr_b,
             zden, bnd_v, s_sh, d_sh, h_sh, gsem, ssem, isem):
    cid = lax.axis_index("c")
    sid = lax.axis_index("s")
    srcb, dstb, exb, hb = [src_a, src_b], [dst_a, dst_b], [ex_a, ex_b], [hr_a, hr_b]

    pltpu.sync_copy(aa_hbm, aa_v)
    pltpu.sync_copy(bnd_hbm, bnd_v)

    braw = bnd_v[0, pl.ds(0, LANES)][0] + bnd_v[1, pl.ds(0, LANES)][0]
    bound = jnp.where(braw > 0.0, braw, 0.2 * braw)

    def _zrow(i, _):
        hr_a[i, :] = jnp.zeros((LANES,), jnp.float32)
        return 0
    lax.fori_loop(0, CHUNK, _zrow, 0)

    def _zden(i, _):
        zden[pl.ds(i * LANES, LANES)] = jnp.zeros((LANES,), jnp.float32)
        return 0
    lax.fori_loop(0, 1024 // LANES, _zden, 0)

    @pl.when(sid < 10)
    def _zero_shared():
        base = sid * 1000
        pltpu.sync_copy(hr_a.at[pl.ds(0, 500)], s_sh.at[pl.ds(base, 500)])
        pltpu.sync_copy(hr_a.at[pl.ds(0, 500)],
                        s_sh.at[pl.ds(base + 500, 500)])
        pltpu.sync_copy(zden.at[pl.ds(0, 1000)], d_sh.at[pl.ds(base, 1000)])

    @pl.when(sid >= 6)
    def _stage_h():
        base = (sid - 6) * 1000
        pltpu.sync_copy(h_hbm.at[pl.ds(base, 1000)],
                        h_sh.at[pl.ds(base, 1000)])

    plsc.subcore_barrier()

    col0 = jnp.zeros((LANES,), jnp.int32)
    col1 = jnp.ones((LANES,), jnp.int32)
    nch = jnp.where(cid == 0, NCH0, NCH1)
    cbase = jnp.where(cid == 0, sid * NCH0, NS * NCH0 + sid * NCH1)
    ebases = [(cbase + k) * CHUNK for k in range(NCH0)]

    def idx_descs(k):
        eb, b = ebases[k], k % 2
        return [
            pltpu.make_async_copy(src_hbm.at[pl.ds(eb, CHUNK)], srcb[b], isem),
            pltpu.make_async_copy(dst_hbm.at[pl.ds(eb, CHUNK)], dstb[b], isem),
        ]

    def gat_descs(k):
        b = k % 2
        return [pltpu.make_async_copy(h_sh.at[srcb[b]], hb[b], gsem)]

    def sc_descs(k):
        b = k % 2
        return [
            pltpu.make_async_copy(hb[b], s_sh.at[dstb[b]], ssem),
            pltpu.make_async_copy(exb[b], d_sh.at[dstb[b]], ssem),
        ]

    def compute_ex(k):
        eb, b = ebases[k], k % 2

        @plsc.parallel_loop(0, CHUNK // LANES, unroll=4)
        def _exbody(i):
            c = i * LANES
            s16 = srcb[b][pl.ds(c, LANES)]
            d16 = dstb[b][pl.ds(c, LANES)]
            e = (plsc.load_gather(aa_v, [s16, col0])
                 + plsc.load_gather(aa_v, [d16, col1]))
            e = jnp.where(e > 0.0, e, 0.2 * e)
            ex = jnp.exp(e - bound)
            eid = eb + c + lax.iota(jnp.int32, 16)
            ex = jnp.where(eid % EPB < EB, ex, 0.0)
            exb[b][pl.ds(c, LANES)] = ex

    def scale(k):
        b = k % 2

        @plsc.parallel_loop(0, CHUNK // LANES, unroll=2)
        def _sbody(g):
            base = g * LANES
            ex16 = exb[b][pl.ds(base, LANES)]
            for l in range(LANES):
                hb[b][base + l, :] = hb[b][base + l, :] * ex16[l]

    for d in idx_descs(0):
        d.start()
    for d in idx_descs(0):
        d.wait()
    for d in gat_descs(0):
        d.start()

    for k in range(NCH0):
        @pl.when(k < nch)
        def _ex(k=k):
            compute_ex(k)

        if k >= 1:
            @pl.when(k - 1 < nch)
            def _drain_sc(k=k):
                for d in sc_descs(k - 1):
                    d.wait()

        if k + 1 < NCH0:
            @pl.when(k + 1 < nch)
            def _fire_idx(k=k):
                for d in idx_descs(k + 1):
                    d.start()

        @pl.when(k < nch)
        def _gath_scale(k=k):
            for d in gat_descs(k):
                d.wait()
            scale(k)

        if k + 1 < NCH0:
            @pl.when(k + 1 < nch)
            def _fire_gat(k=k):
                for d in idx_descs(k + 1):
                    d.wait()
                for d in gat_descs(k + 1):
                    d.start()

        @pl.when(k < nch)
        def _fire_sc(k=k):
            for d in sc_descs(k):
                d.start(add=True)

    @pl.when(NCH0 - 1 < nch)
    def _drain_last():
        for d in sc_descs(NCH0 - 1):
            d.wait()

    plsc.subcore_barrier()

    @pl.when(sid < 10)
    def _copy_out():
        base = sid * 1000
        pltpu.sync_copy(s_sh.at[pl.ds(base, 1000)],
                        s_out.at[pl.ds(cid * N + base, 1000)])
        pltpu.sync_copy(d_sh.at[pl.ds(base, 1000)],
                        d_out.at[pl.ds(cid * N + base, 1000)])


_EXPAND = np.kron(np.eye(8, dtype=np.float32),
                  np.ones((1, D_OUT), np.float32))


def _fin_body(s_ref, d_ref, b_ref, e_ref, o_ref):
    s = s_ref[0] + s_ref[1]
    den8 = d_ref[0] + d_ref[1]
    den = jnp.dot(den8, e_ref[...], preferred_element_type=jnp.float32)
    o_ref[...] = s / (den + 1e-16) + b_ref[...]


_fin = pl.pallas_call(
    _fin_body,
    out_shape=jax.ShapeDtypeStruct((N // 8, 128), jnp.float32),
)


def kernel(x, edge_index, W, att_src, att_dst, bias):
    ei = edge_index.astype(jnp.int32)
    h, asad, bnd, src_p, dst_p = _dense(x, W, att_src, att_dst, ei)
    s_flat, d_flat = _edge_sc(h, asad, bnd, src_p, dst_p)
    out128 = _fin(s_flat.reshape(NC, N // 8, 128),
                  d_flat.reshape(NC, N // 8, 8),
                  jnp.tile(bias, 8).reshape(1, 128),
                  jnp.asarray(_EXPAND))
    return out128.reshape(N, D_OUT)


# speedup vs baseline: 1.0560x; 1.0305x over previous
import jax
import jax.numpy as jnp
import numpy as np
from jax import lax
from jax.experimental import pallas as pl
from jax.experimental.pallas import tpu as pltpu
from jax.experimental.pallas import tpu_sc as plsc

N = 10000
E = 320000
D_IN = 128
D_OUT = 16

NC, NS, LANES = 2, 16, 16
CHUNK = 512
RPC = 4
NCH0, NCH1 = 21, 19
EPAD = NS * (NCH0 + NCH1) * CHUNK
BR = 2000
GRID = N // BR
EB = E // GRID
EPB = EPAD // GRID
PADB = EPB - EB


def _dense_body(x_ref, w_ref, as_ref, ad_ref, ei_ref,
                h_ref, asad_ref, bnd_ref, src_ref, dst_ref):
    i = pl.program_id(0)
    h = lax.dot_general(x_ref[...], w_ref[...], (((1,), (1,)), ((), ())),
                        preferred_element_type=jnp.float32)
    h_ref[...] = h
    att2 = jnp.stack([as_ref[...], ad_ref[...]], axis=1)
    a2 = jnp.dot(h, att2, preferred_element_type=jnp.float32)
    asad_ref[...] = a2
    mas = jnp.max(a2[:, 0])
    mad = jnp.max(a2[:, 1])
    cur = jnp.stack([jnp.full((128,), mas), jnp.full((128,), mad)])

    @pl.when(i == 0)
    def _init():
        bnd_ref[...] = cur

    @pl.when(i > 0)
    def _acc():
        bnd_ref[...] = jnp.maximum(bnd_ref[...], cur)

    src_ref[pl.ds(0, EB)] = ei_ref[0, :]
    src_ref[pl.ds(EB, PADB)] = jnp.zeros((PADB,), jnp.int32)
    dst_ref[pl.ds(0, EB)] = ei_ref[1, :]
    dst_ref[pl.ds(EB, PADB)] = jnp.zeros((PADB,), jnp.int32)


_dense = pl.pallas_call(
    _dense_body,
    grid=(GRID,),
    in_specs=[
        pl.BlockSpec((BR, D_IN), lambda i: (i, 0)),
        pl.BlockSpec((D_OUT, D_IN), lambda i: (0, 0)),
        pl.BlockSpec((D_OUT,), lambda i: (0,)),
        pl.BlockSpec((D_OUT,), lambda i: (0,)),
        pl.BlockSpec((2, EB), lambda i: (0, i)),
    ],
    out_specs=[
        pl.BlockSpec((BR, D_OUT), lambda i: (i, 0)),
        pl.BlockSpec((BR, 2), lambda i: (i, 0)),
        pl.BlockSpec((2, 128), lambda i: (0, 0)),
        pl.BlockSpec((EPB,), lambda i: (i,)),
        pl.BlockSpec((EPB,), lambda i: (i,)),
    ],
    out_shape=[
        jax.ShapeDtypeStruct((N, D_OUT), jnp.float32),
        jax.ShapeDtypeStruct((N, 2), jnp.float32),
        jax.ShapeDtypeStruct((2, 128), jnp.float32),
        jax.ShapeDtypeStruct((EPAD,), jnp.int32),
        jax.ShapeDtypeStruct((EPAD,), jnp.int32),
    ],
)


_mesh = plsc.VectorSubcoreMesh(
    core_axis_name="c", subcore_axis_name="s", num_cores=NC, num_subcores=NS
)


def _sc_kernel_def(fn):
    return pl.kernel(
        fn,
        out_type=(
            jax.ShapeDtypeStruct((NC * N, D_OUT), jnp.float32),
            jax.ShapeDtypeStruct((NC * N,), jnp.float32),
        ),
        mesh=_mesh,
        compiler_params=pltpu.CompilerParams(
            needs_layout_passes=False, use_tc_tiling_on_sc=False
        ),
        scratch_types=[
            pltpu.VMEM((N, 2), jnp.float32),
            pltpu.VMEM((CHUNK,), jnp.int32),
            pltpu.VMEM((CHUNK,), jnp.int32),
            pltpu.VMEM((CHUNK,), jnp.int32),
            pltpu.VMEM((CHUNK,), jnp.int32),
            pltpu.VMEM((CHUNK,), jnp.float32),
            pltpu.VMEM((CHUNK,), jnp.float32),
            pltpu.VMEM((CHUNK, D_OUT), jnp.float32),
            pltpu.VMEM((CHUNK, D_OUT), jnp.float32),
            pltpu.VMEM((1024,), jnp.float32),
            pltpu.VMEM((2, 128), jnp.float32),
            pltpu.VMEM_SHARED((N, D_OUT), jnp.float32),
            pltpu.VMEM_SHARED((N,), jnp.float32),
            pltpu.VMEM_SHARED((N, D_OUT), jnp.float32),
            pltpu.SemaphoreType.DMA,
            pltpu.SemaphoreType.DMA,
            pltpu.SemaphoreType.DMA,
        ],
    )


@_sc_kernel_def
def _edge_sc(h_hbm, aa_hbm, bnd_hbm, src_hbm, dst_hbm, s_out, d_out,
             aa_v, src_a, src_b, dst_a, dst_b, ex_a, ex_b, hr_a, hr_b,
             zden, bnd_v, s_sh, d_sh, h_sh, gsem, ssem, isem):
    cid = lax.axis_index("c")
    sid = lax.axis_index("s")
    srcb, dstb, exb, hb = [src_a, src_b], [dst_a, dst_b], [ex_a, ex_b], [hr_a, hr_b]

    pltpu.sync_copy(aa_hbm, aa_v)
    pltpu.sync_copy(bnd_hbm, bnd_v)

    braw = bnd_v[0, pl.ds(0, LANES)][0] + bnd_v[1, pl.ds(0, LANES)][0]
    bound = jnp.where(braw > 0.0, braw, 0.2 * braw)

    def _zrow(i, _):
        hr_a[i, :] = jnp.zeros((LANES,), jnp.float32)
        return 0
    lax.fori_loop(0, CHUNK, _zrow, 0)

    def _zden(i, _):
        zden[pl.ds(i * LANES, LANES)] = jnp.zeros((LANES,), jnp.float32)
        return 0
    lax.fori_loop(0, 1024 // LANES, _zden, 0)

    @pl.when(sid < 10)
    def _zero_shared():
        base = sid * 1000
        pltpu.sync_copy(hr_a.at[pl.ds(0, 500)], s_sh.at[pl.ds(base, 500)])
        pltpu.sync_copy(hr_a.at[pl.ds(0, 500)],
                        s_sh.at[pl.ds(base + 500, 500)])
        pltpu.sync_copy(zden.at[pl.ds(0, 1000)], d_sh.at[pl.ds(base, 1000)])

    @pl.when(sid >= 6)
    def _stage_h():
        base = (sid - 6) * 1000
        pltpu.sync_copy(h_hbm.at[pl.ds(base, 1000)],
                        h_sh.at[pl.ds(base, 1000)])

    plsc.subcore_barrier()

    col0 = jnp.zeros((LANES,), jnp.int32)
    col1 = jnp.ones((LANES,), jnp.int32)
    nch = jnp.where(cid == 0, NCH0, NCH1)
    cbase = jnp.where(cid == 0, sid * NCH0, NS * NCH0 + sid * NCH1)
    ebases = [(cbase + k) * CHUNK for k in range(NCH0)]

    def idx_descs(k):
        eb, b = ebases[k], k % 2
        return [
            pltpu.make_async_copy(src_hbm.at[pl.ds(eb, CHUNK)], srcb[b], isem),
            pltpu.make_async_copy(dst_hbm.at[pl.ds(eb, CHUNK)], dstb[b], isem),
        ]

    def gat_descs(k):
        b = k % 2
        return [pltpu.make_async_copy(h_sh.at[srcb[b]], hb[b], gsem)]

    def sc_descs(k):
        b = k % 2
        return [
            pltpu.make_async_copy(hb[b], s_sh.at[dstb[b]], ssem),
            pltpu.make_async_copy(exb[b], d_sh.at[dstb[b]], ssem),
        ]

    def compute_ex(k):
        eb, b = ebases[k], k % 2

        def _exbody(i, _):
            c = i * LANES
            s16 = srcb[b][pl.ds(c, LANES)]
            d16 = dstb[b][pl.ds(c, LANES)]
            e = (plsc.load_gather(aa_v, [s16, col0])
                 + plsc.load_gather(aa_v, [d16, col1]))
            e = jnp.where(e > 0.0, e, 0.2 * e)
            ex = jnp.exp(e - bound)
            eid = eb + c + lax.iota(jnp.int32, 16)
            ex = jnp.where(eid % EPB < EB, ex, 0.0)
            exb[b][pl.ds(c, LANES)] = ex
            return 0
        lax.fori_loop(0, CHUNK // LANES, _exbody, 0)

    def scale(k):
        b = k % 2

        def _sbody(g, _):
            base = g * LANES
            ex16 = exb[b][pl.ds(base, LANES)]
            for l in range(LANES):
                hb[b][base + l, :] = hb[b][base + l, :] * ex16[l]
            return 0
        lax.fori_loop(0, CHUNK // LANES, _sbody, 0)

    for d in idx_descs(0):
        d.start()
    for d in idx_descs(0):
        d.wait()
    for d in gat_descs(0):
        d.start()

    for k in range(NCH0):
        @pl.when(k < nch)
        def _ex(k=k):
            compute_ex(k)

        if k >= 1:
            @pl.when(k - 1 < nch)
            def _drain_sc(k=k):
                for d in sc_descs(k - 1):
                    d.wait()

        if k + 1 < NCH0:
            @pl.when(k + 1 < nch)
            def _fire_idx(k=k):
                for d in idx_descs(k + 1):
                    d.start()

        @pl.when(k < nch)
        def _gath_scale(k=k):
            for d in gat_descs(k):
                d.wait()
            scale(k)

        if k + 1 < NCH0:
            @pl.when(k + 1 < nch)
            def _fire_gat(k=k):
                for d in idx_descs(k + 1):
                    d.wait()
                for d in gat_descs(k + 1):
                    d.start()

        @pl.when(k < nch)
        def _fire_sc(k=k):
            for d in sc_descs(k):
                d.start(add=True)

    @pl.when(NCH0 - 1 < nch)
    def _drain_last():
        for d in sc_descs(NCH0 - 1):
            d.wait()

    plsc.subcore_barrier()

    @pl.when(sid < 10)
    def _copy_out():
        base = sid * 1000
        pltpu.sync_copy(s_sh.at[pl.ds(base, 1000)],
                        s_out.at[pl.ds(cid * N + base, 1000)])
        pltpu.sync_copy(d_sh.at[pl.ds(base, 1000)],
                        d_out.at[pl.ds(cid * N + base, 1000)])


_EXPAND = np.kron(np.eye(8, dtype=np.float32),
                  np.ones((1, D_OUT), np.float32))


def _fin_body(s_ref, d_ref, b_ref, e_ref, o_ref):
    s = s_ref[0] + s_ref[1]
    den8 = d_ref[0] + d_ref[1]
    den = jnp.dot(den8, e_ref[...], preferred_element_type=jnp.float32)
    o_ref[...] = s / (den + 1e-16) + b_ref[...]


_fin = pl.pallas_call(
    _fin_body,
    out_shape=jax.ShapeDtypeStruct((N // 8, 128), jnp.float32),
)


def kernel(x, edge_index, W, att_src, att_dst, bias):
    ei = edge_index.astype(jnp.int32)
    h, asad, bnd, src_p, dst_p = _dense(x, W.T, att_src, att_dst, ei)
    s_flat, d_flat = _edge_sc(h, asad, bnd, src_p, dst_p)
    out128 = _fin(s_flat.reshape(NC, N // 8, 128),
                  d_flat.reshape(NC, N // 8, 8),
                  jnp.tile(bias, 8).reshape(1, 128),
                  jnp.asarray(_EXPAND))
    return out128.reshape(N, D_OUT)
